# baseline (speedup 1.0000x reference)
import functools

import jax
import jax.numpy as jnp
from jax import lax
from jax.experimental import pallas as pl
from jax.experimental.pallas import tpu as pltpu
from jax.experimental.pallas import tpu_sc as plsc

_C = 8


def _sc_gather2(tt_t, ct_t, target_idx, context_idx):
    B = target_idx.shape[0]
    D = tt_t.shape[0]
    info = plsc.get_sparse_core_info()
    nw = info.num_cores * info.num_subcores
    b_per_w = B // nw
    n_chunks = b_per_w // _C
    rows_per_w = b_per_w // 8

    mesh = plsc.VectorSubcoreMesh(core_axis_name="c", subcore_axis_name="s")

    @functools.partial(
        pl.kernel,
        mesh=mesh,
        out_type=[
            jax.ShapeDtypeStruct((B // 8, 8 * D), jnp.float32),
            jax.ShapeDtypeStruct((B // 8, 8 * D), jnp.float32),
        ],
        scratch_types=[
            pltpu.VMEM((b_per_w,), jnp.int32),
            pltpu.VMEM((b_per_w,), jnp.int32),
            pltpu.VMEM((_C * D, 128), jnp.float32),
            pltpu.VMEM((_C * D, 128), jnp.float32),
            pltpu.VMEM((_C * D, 128), jnp.float32),
            pltpu.VMEM((_C * D, 128), jnp.float32),
            pltpu.VMEM((rows_per_w, 8 * D), jnp.float32),
            pltpu.VMEM((rows_per_w, 8 * D), jnp.float32),
            pltpu.SemaphoreType.DMA,
            pltpu.SemaphoreType.DMA,
            pltpu.SemaphoreType.DMA,
            pltpu.SemaphoreType.DMA,
        ],
        compiler_params=pltpu.CompilerParams(needs_layout_passes=False),
    )
    def gather_kernel(tt_hbm, ct_hbm, ti_hbm, ci_hbm, t_out, c_out,
                      ti_v, ci_v, t_tiles_e, t_tiles_o, c_tiles_e, c_tiles_o,
                      t_pack, c_pack, sem_te, sem_to, sem_ce, sem_co):
        wid = lax.axis_index("s") * info.num_cores + lax.axis_index("c")
        base = wid * b_per_w
        pltpu.sync_copy(ti_hbm.at[pl.ds(base, b_per_w)], ti_v)
        pltpu.sync_copy(ci_hbm.at[pl.ds(base, b_per_w)], ci_v)
        iota16 = lax.iota(jnp.int32, 16)

        def fire(tcb, ccb, half, t_tiles, c_tiles, sem_t, sem_c):
            copies = []
            for s in range(_C):
                l = half * _C + s
                copies.append(pltpu.async_copy(
                    tt_hbm.at[:, pl.ds(pl.multiple_of(tcb[l], 128), 128)],
                    t_tiles.at[pl.ds(s * D, D)], sem_t))
                copies.append(pltpu.async_copy(
                    ct_hbm.at[:, pl.ds(pl.multiple_of(ccb[l], 128), 128)],
                    c_tiles.at[pl.ds(s * D, D)], sem_c))
            return copies

        def extract(tlane, clane, half, prow, t_tiles, c_tiles):
            for s in range(_C):
                l = half * _C + s
                rv = iota16 + s * D
                tvals = plsc.load_gather(
                    t_tiles, [rv, jnp.full((16,), 1, jnp.int32) * tlane[l]])
                cvals = plsc.load_gather(
                    c_tiles, [rv, jnp.full((16,), 1, jnp.int32) * clane[l]])
                t_pack[prow + s // 8, pl.ds((s % 8) * D, D)] = tvals
                c_pack[prow + s // 8, pl.ds((s % 8) * D, D)] = cvals

        def pair_body(g2, _):
            tidx = ti_v[pl.ds(g2 * 16, 16)]
            cidx = ci_v[pl.ds(g2 * 16, 16)]
            tcb = lax.bitwise_and(tidx, ~127)
            ccb = lax.bitwise_and(cidx, ~127)
            tlane = lax.bitwise_and(tidx, 127)
            clane = lax.bitwise_and(cidx, 127)
            prow = g2 * 2
            cps_a = fire(tcb, ccb, 0, t_tiles_e, c_tiles_e, sem_te, sem_ce)
            cps_b = fire(tcb, ccb, 1, t_tiles_o, c_tiles_o, sem_to, sem_co)
            for cp in cps_a:
                cp.wait()
            extract(tlane, clane, 0, prow, t_tiles_e, c_tiles_e)
            for cp in cps_b:
                cp.wait()
            extract(tlane, clane, 1, prow + 1, t_tiles_o, c_tiles_o)
            return ()

        lax.fori_loop(0, n_chunks // 2, pair_body, ())
        pltpu.sync_copy(t_pack, t_out.at[pl.ds(wid * rows_per_w, rows_per_w)])
        pltpu.sync_copy(c_pack, c_out.at[pl.ds(wid * rows_per_w, rows_per_w)])

    return gather_kernel(tt_t, ct_t, target_idx, context_idx)


def _mlp_body(t_ref, c_ref, w1a_ref, w1b_ref, b1_ref, w2_ref, b2_ref, o_ref):
    h = (
        jnp.dot(t_ref[...], w1a_ref[...], preferred_element_type=jnp.float32)
        + jnp.dot(c_ref[...], w1b_ref[...], preferred_element_type=jnp.float32)
        + b1_ref[...]
    )
    h = jnp.maximum(h, 0.0)
    o_ref[...] = jax.nn.sigmoid(
        jnp.dot(h, w2_ref[...], preferred_element_type=jnp.float32)
        + b2_ref[...])


def _tc_mlp(t_pack, c_pack, W1, b1, W2, b2):
    n, lanes = t_pack.shape
    D = lanes // 8
    eye8 = jnp.eye(8, dtype=jnp.float32)
    w1a_bd = jnp.kron(eye8, W1[:D, :])
    w1b_bd = jnp.kron(eye8, W1[D:, :])
    b1_tiled = jnp.tile(b1, 8).reshape(1, lanes)
    w2_bd = jnp.kron(eye8, W2)
    out = pl.pallas_call(
        _mlp_body,
        out_shape=jax.ShapeDtypeStruct((n, 8), jnp.float32),
    )(t_pack, c_pack, w1a_bd, w1b_bd, b1_tiled, w2_bd, b2.reshape(1, 1))
    return out


def kernel(target, context, target_table, context_table, W1, b1, W2, b2):
    target = target.astype(jnp.int32)
    context = context.astype(jnp.int32)
    B = target.shape[0]
    t_pack, c_pack = _sc_gather2(
        target_table.T, context_table.T, target, context)
    out = _tc_mlp(t_pack, c_pack, W1, b1, W2, b2)
    return out.reshape(B, 1)


# speedup vs baseline: 1.0078x; 1.0078x over previous
import functools

import jax
import jax.numpy as jnp
from jax import lax
from jax.experimental import pallas as pl
from jax.experimental.pallas import tpu as pltpu
from jax.experimental.pallas import tpu_sc as plsc

_C = 16


def _sc_gather2(tt_t, ct_t, target_idx, context_idx):
    B = target_idx.shape[0]
    D = tt_t.shape[0]
    info = plsc.get_sparse_core_info()
    nw = info.num_cores * info.num_subcores
    b_per_w = B // nw
    n_chunks = b_per_w // _C
    rows_per_w = b_per_w // 8

    mesh = plsc.VectorSubcoreMesh(core_axis_name="c", subcore_axis_name="s")

    @functools.partial(
        pl.kernel,
        mesh=mesh,
        out_type=[
            jax.ShapeDtypeStruct((B // 8, 8 * D), jnp.float32),
            jax.ShapeDtypeStruct((B // 8, 8 * D), jnp.float32),
        ],
        scratch_types=[
            pltpu.VMEM((b_per_w,), jnp.int32),
            pltpu.VMEM((b_per_w,), jnp.int32),
            pltpu.VMEM((_C * D, 128), jnp.float32),
            pltpu.VMEM((_C * D, 128), jnp.float32),
            pltpu.VMEM((rows_per_w, 8 * D), jnp.float32),
            pltpu.VMEM((rows_per_w, 8 * D), jnp.float32),
            pltpu.SemaphoreType.DMA,
            pltpu.SemaphoreType.DMA,
        ],
        compiler_params=pltpu.CompilerParams(needs_layout_passes=False),
    )
    def gather_kernel(tt_hbm, ct_hbm, ti_hbm, ci_hbm, t_out, c_out,
                      ti_v, ci_v, t_tiles, c_tiles,
                      t_pack, c_pack, sem_t, sem_c):
        wid = lax.axis_index("s") * info.num_cores + lax.axis_index("c")
        base = wid * b_per_w
        pltpu.sync_copy(ti_hbm.at[pl.ds(base, b_per_w)], ti_v)
        pltpu.sync_copy(ci_hbm.at[pl.ds(base, b_per_w)], ci_v)
        iota16 = lax.iota(jnp.int32, 16)

        def chunk_body(chunk, _):
            tidx = ti_v[pl.ds(chunk * _C, _C)]
            cidx = ci_v[pl.ds(chunk * _C, _C)]
            tcb = lax.bitwise_and(tidx, ~127)
            ccb = lax.bitwise_and(cidx, ~127)
            t_copies = []
            c_copies = []
            for s in range(_C):
                t_copies.append(pltpu.async_copy(
                    tt_hbm.at[:, pl.ds(pl.multiple_of(tcb[s], 128), 128)],
                    t_tiles.at[pl.ds(s * D, D)], sem_t))
                c_copies.append(pltpu.async_copy(
                    ct_hbm.at[:, pl.ds(pl.multiple_of(ccb[s], 128), 128)],
                    c_tiles.at[pl.ds(s * D, D)], sem_c))
            tlane = lax.bitwise_and(tidx, 127)
            clane = lax.bitwise_and(cidx, 127)
            prow = chunk * (_C // 8)
            for cp in t_copies:
                cp.wait()
            for s in range(_C):
                rv = iota16 + s * D
                tvals = plsc.load_gather(
                    t_tiles, [rv, jnp.full((16,), 1, jnp.int32) * tlane[s]])
                t_pack[prow + s // 8, pl.ds((s % 8) * D, D)] = tvals
            for cp in c_copies:
                cp.wait()
            for s in range(_C):
                rv = iota16 + s * D
                cvals = plsc.load_gather(
                    c_tiles, [rv, jnp.full((16,), 1, jnp.int32) * clane[s]])
                c_pack[prow + s // 8, pl.ds((s % 8) * D, D)] = cvals
            return ()

        lax.fori_loop(0, n_chunks, chunk_body, ())
        pltpu.sync_copy(t_pack, t_out.at[pl.ds(wid * rows_per_w, rows_per_w)])
        pltpu.sync_copy(c_pack, c_out.at[pl.ds(wid * rows_per_w, rows_per_w)])

    return gather_kernel(tt_t, ct_t, target_idx, context_idx)


def _mlp_body(t_ref, c_ref, w1a_ref, w1b_ref, b1_ref, w2_ref, b2_ref, o_ref):
    h = (
        jnp.dot(t_ref[...], w1a_ref[...], preferred_element_type=jnp.float32)
        + jnp.dot(c_ref[...], w1b_ref[...], preferred_element_type=jnp.float32)
        + b1_ref[...]
    )
    h = jnp.maximum(h, 0.0)
    o_ref[...] = jax.nn.sigmoid(
        jnp.dot(h, w2_ref[...], preferred_element_type=jnp.float32)
        + b2_ref[...])


def _tc_mlp(t_pack, c_pack, W1, b1, W2, b2):
    n, lanes = t_pack.shape
    D = lanes // 8
    eye8 = jnp.eye(8, dtype=jnp.float32)
    w1a_bd = jnp.kron(eye8, W1[:D, :])
    w1b_bd = jnp.kron(eye8, W1[D:, :])
    b1_tiled = jnp.tile(b1, 8).reshape(1, lanes)
    w2_bd = jnp.kron(eye8, W2)
    out = pl.pallas_call(
        _mlp_body,
        out_shape=jax.ShapeDtypeStruct((n, 8), jnp.float32),
    )(t_pack, c_pack, w1a_bd, w1b_bd, b1_tiled, w2_bd, b2.reshape(1, 1))
    return out


def kernel(target, context, target_table, context_table, W1, b1, W2, b2):
    target = target.astype(jnp.int32)
    context = context.astype(jnp.int32)
    B = target.shape[0]
    t_pack, c_pack = _sc_gather2(
        target_table.T, context_table.T, target, context)
    out = _tc_mlp(t_pack, c_pack, W1, b1, W2, b2)
    return out.reshape(B, 1)


# speedup vs baseline: 1.0140x; 1.0062x over previous
import functools

import jax
import jax.numpy as jnp
from jax import lax
from jax.experimental import pallas as pl
from jax.experimental.pallas import tpu as pltpu
from jax.experimental.pallas import tpu_sc as plsc

_C = 16


def _sc_gather2(tt_t, ct_t, target_idx, context_idx):
    B = target_idx.shape[0]
    D = tt_t.shape[0]
    info = plsc.get_sparse_core_info()
    nw = info.num_cores * info.num_subcores
    b_per_w = B // nw
    n_chunks = b_per_w // _C
    rows_per_w = b_per_w // 8

    mesh = plsc.VectorSubcoreMesh(core_axis_name="c", subcore_axis_name="s")

    @functools.partial(
        pl.kernel,
        mesh=mesh,
        out_type=[
            jax.ShapeDtypeStruct((B // 8, 8 * D), jnp.float32),
            jax.ShapeDtypeStruct((B // 8, 8 * D), jnp.float32),
        ],
        scratch_types=[
            pltpu.VMEM((b_per_w,), jnp.int32),
            pltpu.VMEM((b_per_w,), jnp.int32),
            pltpu.VMEM((_C * D, 128), jnp.float32),
            pltpu.VMEM((_C * D, 128), jnp.float32),
            pltpu.VMEM((rows_per_w, 8 * D), jnp.float32),
            pltpu.VMEM((rows_per_w, 8 * D), jnp.float32),
            pltpu.SemaphoreType.DMA,
            pltpu.SemaphoreType.DMA,
        ],
        compiler_params=pltpu.CompilerParams(needs_layout_passes=False),
    )
    def gather_kernel(tt_hbm, ct_hbm, ti_hbm, ci_hbm, t_out, c_out,
                      ti_v, ci_v, t_tiles, c_tiles,
                      t_pack, c_pack, sem_t, sem_c):
        wid = lax.axis_index("s") * info.num_cores + lax.axis_index("c")
        base = wid * b_per_w
        pltpu.sync_copy(ti_hbm.at[pl.ds(base, b_per_w)], ti_v)
        pltpu.sync_copy(ci_hbm.at[pl.ds(base, b_per_w)], ci_v)
        iota16 = lax.iota(jnp.int32, 16)

        def chunk_body(chunk, _):
            tidx = ti_v[pl.ds(chunk * _C, _C)]
            cidx = ci_v[pl.ds(chunk * _C, _C)]
            tcb = lax.bitwise_and(tidx, ~127)
            ccb = lax.bitwise_and(cidx, ~127)
            t_copies = []
            c_copies = []
            for s in range(_C):
                t_copies.append(pltpu.async_copy(
                    tt_hbm.at[:, pl.ds(pl.multiple_of(tcb[s], 128), 128)],
                    t_tiles.at[pl.ds(s * D, D)], sem_t))
                c_copies.append(pltpu.async_copy(
                    ct_hbm.at[:, pl.ds(pl.multiple_of(ccb[s], 128), 128)],
                    c_tiles.at[pl.ds(s * D, D)], sem_c))
            tlane = lax.bitwise_and(tidx, 127)
            clane = lax.bitwise_and(cidx, 127)
            prow = chunk * (_C // 8)
            for cp in t_copies:
                cp.wait()
            for cp in c_copies:
                cp.wait()
            for s in range(_C):
                rv = iota16 + s * D
                tvals = plsc.load_gather(
                    t_tiles, [rv, jnp.full((16,), 1, jnp.int32) * tlane[s]])
                cvals = plsc.load_gather(
                    c_tiles, [rv, jnp.full((16,), 1, jnp.int32) * clane[s]])
                t_pack[prow + s // 8, pl.ds((s % 8) * D, D)] = tvals
                c_pack[prow + s // 8, pl.ds((s % 8) * D, D)] = cvals
            return ()

        lax.fori_loop(0, n_chunks, chunk_body, ())
        pltpu.sync_copy(t_pack, t_out.at[pl.ds(wid * rows_per_w, rows_per_w)])
        pltpu.sync_copy(c_pack, c_out.at[pl.ds(wid * rows_per_w, rows_per_w)])

    return gather_kernel(tt_t, ct_t, target_idx, context_idx)


def _mlp_body(t_ref, c_ref, w1a_ref, w1b_ref, b1_ref, w2_ref, b2_ref, o_ref):
    h = (
        jnp.dot(t_ref[...], w1a_ref[...], preferred_element_type=jnp.float32)
        + jnp.dot(c_ref[...], w1b_ref[...], preferred_element_type=jnp.float32)
        + b1_ref[...]
    )
    h = jnp.maximum(h, 0.0)
    o_ref[...] = jax.nn.sigmoid(
        jnp.dot(h, w2_ref[...], preferred_element_type=jnp.float32)
        + b2_ref[...])


def _tc_mlp(t_pack, c_pack, W1, b1, W2, b2):
    n, lanes = t_pack.shape
    D = lanes // 8
    eye8 = jnp.eye(8, dtype=jnp.float32)
    w1a_bd = jnp.kron(eye8, W1[:D, :])
    w1b_bd = jnp.kron(eye8, W1[D:, :])
    b1_tiled = jnp.tile(b1, 8).reshape(1, lanes)
    w2_bd = jnp.kron(eye8, W2)
    out = pl.pallas_call(
        _mlp_body,
        out_shape=jax.ShapeDtypeStruct((n, 8), jnp.float32),
    )(t_pack, c_pack, w1a_bd, w1b_bd, b1_tiled, w2_bd, b2.reshape(1, 1))
    return out


def kernel(target, context, target_table, context_table, W1, b1, W2, b2):
    target = target.astype(jnp.int32)
    context = context.astype(jnp.int32)
    B = target.shape[0]
    t_pack, c_pack = _sc_gather2(
        target_table.T, context_table.T, target, context)
    out = _tc_mlp(t_pack, c_pack, W1, b1, W2, b2)
    return out.reshape(B, 1)
